# baseline (speedup 1.0000x reference)
import functools

import jax
import jax.numpy as jnp
from jax import lax
from jax.experimental import pallas as pl
from jax.experimental.pallas import tpu as pltpu
from jax.experimental.pallas import tpu_sc as plsc

_NUM_CODES = 1024
_CODE_DIM = 64
_COMMITMENT = 0.25
_TT = 1024


def _dist_argmin_body(xt_ref, emb_ref, idx_ref, acc_ref, *, loss_scale):
    xt = xt_ref[...]
    emb = emb_ref[...]
    prod = lax.dot_general(
        emb, xt, (((1,), (0,)), ((), ())),
        preferred_element_type=jnp.float32,
        precision=lax.Precision.DEFAULT,
    )
    x2 = jnp.sum(xt * xt, axis=0, keepdims=True)
    e2 = jnp.sum(emb * emb, axis=1, keepdims=True)
    dist = x2 - 2.0 * prod + e2
    m = jnp.min(dist, axis=0, keepdims=True)
    ids = lax.broadcasted_iota(jnp.int32, dist.shape, 0)
    idx = jnp.min(jnp.where(dist == m, ids, jnp.int32(2**30)), axis=0)
    idx_ref[...] = idx.reshape(1, 1, _TT)

    @pl.when(pl.program_id(0) == 0)
    def _():
        acc_ref[...] = jnp.zeros((1, 1), jnp.float32)

    acc_ref[...] += (jnp.sum(m) * loss_scale).reshape(1, 1)


def _dist_argmin(xt2d, emb, loss_scale):
    n_slab = xt2d.shape[0] // _CODE_DIM
    body = functools.partial(_dist_argmin_body, loss_scale=loss_scale)
    return pl.pallas_call(
        body,
        grid=(n_slab,),
        in_specs=[
            pl.BlockSpec((_CODE_DIM, _TT), lambda i: (i, 0)),
            pl.BlockSpec((_NUM_CODES, _CODE_DIM), lambda i: (0, 0)),
        ],
        out_specs=[
            pl.BlockSpec((1, 1, _TT), lambda i: (i, 0, 0)),
            pl.BlockSpec((1, 1), lambda i: (0, 0)),
        ],
        out_shape=[
            jax.ShapeDtypeStruct((n_slab, 1, _TT), jnp.int32),
            jax.ShapeDtypeStruct((1, 1), jnp.float32),
        ],
    )(xt2d, emb)


def _make_sc_gather_t(n_tok, width):
    info = plsc.get_sparse_core_info()
    nc, ns, nl = info.num_cores, info.num_subcores, info.num_lanes
    nw = nc * ns
    b_per_w = n_tok // nw
    d_half = width // 2
    mesh = plsc.VectorSubcoreMesh(core_axis_name="c", subcore_axis_name="s")

    n_pass = 4
    d_pp = width // n_pass

    @functools.partial(
        pl.kernel,
        mesh=mesh,
        compiler_params=pltpu.CompilerParams(
            use_tc_tiling_on_sc=False, needs_layout_passes=False),
        out_type=jax.ShapeDtypeStruct((nw * width, b_per_w), jnp.float32),
        scratch_types=[
            pltpu.VMEM((width, _NUM_CODES), jnp.float32),
            pltpu.VMEM((b_per_w,), jnp.int32),
            pltpu.VMEM((d_pp, b_per_w), jnp.float32),
            pltpu.VMEM((d_pp, b_per_w), jnp.float32),
            pltpu.SemaphoreType.DMA,
            pltpu.SemaphoreType.DMA,
            pltpu.SemaphoreType.DMA,
        ],
    )
    def gather_k(tab_hbm, idx_hbm, out_hbm, tab_v, idx_v, o0, o1, si, s0, s1):
        wid = lax.axis_index("s") * nc + lax.axis_index("c")
        ht = pltpu.async_copy(tab_hbm, tab_v, si)
        hi = pltpu.async_copy(
            idx_hbm.at[pl.ds(wid * b_per_w, b_per_w)], idx_v, si)
        ht.wait()
        hi.wait()
        outs, sems = (o0, o1), (s0, s1)
        waits = [None] * n_pass
        for p in range(n_pass):
            if p >= 2:
                waits[p - 2].wait()
            out_v = outs[p % 2]
            rows = [jnp.full((nl,), p * d_pp + d, jnp.int32)
                    for d in range(d_pp)]

            def tok_body(t, _, out_v=out_v, rows=rows):
                iv = idx_v[pl.ds(t * nl, nl)]
                for d in range(d_pp):
                    out_v[d, pl.ds(t * nl, nl)] = plsc.load_gather(
                        tab_v, [rows[d], iv])
                return 0

            lax.fori_loop(0, b_per_w // nl, tok_body, 0, unroll=4)
            waits[p] = pltpu.async_copy(
                out_v, out_hbm.at[pl.ds(wid * width + p * d_pp, d_pp)],
                sems[p % 2])
        waits[n_pass - 2].wait()
        waits[n_pass - 1].wait()

    return gather_k


def kernel(z, embedding):
    n_tok = z.shape[0] * z.shape[1]
    scale = _COMMITMENT / (n_tok * _CODE_DIM)
    xt2d = jnp.transpose(z, (0, 2, 1)).reshape(-1, z.shape[1])
    idx3d, loss2d = _dist_argmin(xt2d, embedding, scale)
    indices = idx3d.reshape(n_tok)
    embt = jnp.transpose(embedding)
    zq_t = _make_sc_gather_t(n_tok, _CODE_DIM)(embt, indices)
    z_q = zq_t.reshape(z.shape[0], _CODE_DIM, z.shape[1]).transpose(0, 2, 1)
    return z_q, loss2d.reshape(()), indices


# speedup vs baseline: 1.0275x; 1.0275x over previous
import functools

import jax
import jax.numpy as jnp
from jax import lax
from jax.experimental import pallas as pl
from jax.experimental.pallas import tpu as pltpu
from jax.experimental.pallas import tpu_sc as plsc

_NUM_CODES = 1024
_CODE_DIM = 64
_COMMITMENT = 0.25
_TT = 1024


def _dist_argmin_body(xt_ref, emb_ref, idx_ref, acc_ref, *, loss_scale):
    xt = xt_ref[...]
    emb = emb_ref[...]
    prod = lax.dot_general(
        emb, xt, (((1,), (0,)), ((), ())),
        preferred_element_type=jnp.float32,
        precision=lax.Precision.DEFAULT,
    )
    x2 = jnp.sum(xt * xt, axis=0, keepdims=True)
    e2 = jnp.sum(emb * emb, axis=1, keepdims=True)
    nch = 8
    ch = _NUM_CODES // nch
    ids_f = lax.broadcasted_iota(jnp.int32, (ch, _TT), 0).astype(jnp.float32)
    big = jnp.float32(2.0e9)
    run_m = None
    for c in range(nch):
        dc = (x2 - 2.0 * prod[c * ch:(c + 1) * ch, :]
              + e2[c * ch:(c + 1) * ch, :])
        mc = jnp.min(dc, axis=0, keepdims=True)
        loc = jnp.min(jnp.where(dc == mc, ids_f, big), axis=0, keepdims=True)
        if run_m is None:
            run_m, run_loc = mc, loc
            run_c = jnp.zeros((1, _TT), jnp.float32)
        else:
            upd = mc < run_m
            run_loc = jnp.where(upd, loc, run_loc)
            run_c = jnp.where(upd, jnp.float32(c), run_c)
            run_m = jnp.minimum(run_m, mc)
    m = run_m
    idx = (run_c * ch + run_loc).astype(jnp.int32)
    idx_ref[...] = idx.reshape(1, 1, _TT)

    @pl.when(pl.program_id(0) == 0)
    def _():
        acc_ref[...] = jnp.zeros((1, 1), jnp.float32)

    acc_ref[...] += (jnp.sum(m) * loss_scale).reshape(1, 1)


def _dist_argmin(xt2d, emb, loss_scale):
    n_slab = xt2d.shape[0] // _CODE_DIM
    body = functools.partial(_dist_argmin_body, loss_scale=loss_scale)
    return pl.pallas_call(
        body,
        grid=(n_slab,),
        in_specs=[
            pl.BlockSpec((_CODE_DIM, _TT), lambda i: (i, 0)),
            pl.BlockSpec((_NUM_CODES, _CODE_DIM), lambda i: (0, 0)),
        ],
        out_specs=[
            pl.BlockSpec((1, 1, _TT), lambda i: (i, 0, 0)),
            pl.BlockSpec((1, 1), lambda i: (0, 0)),
        ],
        out_shape=[
            jax.ShapeDtypeStruct((n_slab, 1, _TT), jnp.int32),
            jax.ShapeDtypeStruct((1, 1), jnp.float32),
        ],
    )(xt2d, emb)


def _make_sc_gather_t(n_tok, width):
    info = plsc.get_sparse_core_info()
    nc, ns, nl = info.num_cores, info.num_subcores, info.num_lanes
    nw = nc * ns
    b_per_w = n_tok // nw
    d_half = width // 2
    mesh = plsc.VectorSubcoreMesh(core_axis_name="c", subcore_axis_name="s")

    n_pass = 4
    d_pp = width // n_pass

    @functools.partial(
        pl.kernel,
        mesh=mesh,
        compiler_params=pltpu.CompilerParams(
            use_tc_tiling_on_sc=False, needs_layout_passes=False),
        out_type=jax.ShapeDtypeStruct((nw * width, b_per_w), jnp.float32),
        scratch_types=[
            pltpu.VMEM((width, _NUM_CODES), jnp.float32),
            pltpu.VMEM((b_per_w,), jnp.int32),
            pltpu.VMEM((d_pp, b_per_w), jnp.float32),
            pltpu.VMEM((d_pp, b_per_w), jnp.float32),
            pltpu.SemaphoreType.DMA,
            pltpu.SemaphoreType.DMA,
            pltpu.SemaphoreType.DMA,
        ],
    )
    def gather_k(tab_hbm, idx_hbm, out_hbm, tab_v, idx_v, o0, o1, si, s0, s1):
        wid = lax.axis_index("s") * nc + lax.axis_index("c")
        ht = pltpu.async_copy(tab_hbm, tab_v, si)
        hi = pltpu.async_copy(
            idx_hbm.at[pl.ds(wid * b_per_w, b_per_w)], idx_v, si)
        ht.wait()
        hi.wait()
        outs, sems = (o0, o1), (s0, s1)
        waits = [None] * n_pass
        for p in range(n_pass):
            if p >= 2:
                waits[p - 2].wait()
            out_v = outs[p % 2]
            rows = [jnp.full((nl,), p * d_pp + d, jnp.int32)
                    for d in range(d_pp)]

            def tok_body(t, _, out_v=out_v, rows=rows):
                iv = idx_v[pl.ds(t * nl, nl)]
                for d in range(d_pp):
                    out_v[d, pl.ds(t * nl, nl)] = plsc.load_gather(
                        tab_v, [rows[d], iv])
                return 0

            lax.fori_loop(0, b_per_w // nl, tok_body, 0, unroll=False)
            waits[p] = pltpu.async_copy(
                out_v, out_hbm.at[pl.ds(wid * width + p * d_pp, d_pp)],
                sems[p % 2])
        waits[n_pass - 2].wait()
        waits[n_pass - 1].wait()

    return gather_k


def kernel(z, embedding):
    n_tok = z.shape[0] * z.shape[1]
    scale = _COMMITMENT / (n_tok * _CODE_DIM)
    xt2d = jnp.transpose(z, (0, 2, 1)).reshape(-1, z.shape[1])
    idx3d, loss2d = _dist_argmin(xt2d, embedding, scale)
    indices = idx3d.reshape(n_tok)
    embt = jnp.transpose(embedding)
    zq_t = _make_sc_gather_t(n_tok, _CODE_DIM)(embt, indices)
    z_q = zq_t.reshape(z.shape[0], _CODE_DIM, z.shape[1]).transpose(0, 2, 1)
    return z_q, loss2d.reshape(()), indices


# speedup vs baseline: 1.0417x; 1.0138x over previous
import functools

import jax
import jax.numpy as jnp
from jax import lax
from jax.experimental import pallas as pl
from jax.experimental.pallas import tpu as pltpu
from jax.experimental.pallas import tpu_sc as plsc

_NUM_CODES = 1024
_CODE_DIM = 64
_COMMITMENT = 0.25
_TT = 1024


def _dist_argmin_body(xt_ref, emb_ref, idx_ref, acc_ref, *, loss_scale):
    xt = xt_ref[...]
    emb = emb_ref[...]
    prod = lax.dot_general(
        emb, xt, (((1,), (0,)), ((), ())),
        preferred_element_type=jnp.float32,
        precision=lax.Precision.DEFAULT,
    )
    x2 = jnp.sum(xt * xt, axis=0, keepdims=True)
    e2 = jnp.sum(emb * emb, axis=1, keepdims=True)
    nch = 4
    ch = _NUM_CODES // nch
    ids_f = lax.broadcasted_iota(jnp.int32, (ch, _TT), 0).astype(jnp.float32)
    big = jnp.float32(2.0e9)
    run_m = None
    for c in range(nch):
        dc = (x2 - 2.0 * prod[c * ch:(c + 1) * ch, :]
              + e2[c * ch:(c + 1) * ch, :])
        mc = jnp.min(dc, axis=0, keepdims=True)
        loc = jnp.min(jnp.where(dc == mc, ids_f, big), axis=0, keepdims=True)
        if run_m is None:
            run_m, run_loc = mc, loc
            run_c = jnp.zeros((1, _TT), jnp.float32)
        else:
            upd = mc < run_m
            run_loc = jnp.where(upd, loc, run_loc)
            run_c = jnp.where(upd, jnp.float32(c), run_c)
            run_m = jnp.minimum(run_m, mc)
    m = run_m
    idx = (run_c * ch + run_loc).astype(jnp.int32)
    idx_ref[...] = idx.reshape(1, 1, _TT)

    @pl.when(pl.program_id(0) == 0)
    def _():
        acc_ref[...] = jnp.zeros((1, 1), jnp.float32)

    acc_ref[...] += (jnp.sum(m) * loss_scale).reshape(1, 1)


def _dist_argmin(xt2d, emb, loss_scale):
    n_slab = xt2d.shape[0] // _CODE_DIM
    body = functools.partial(_dist_argmin_body, loss_scale=loss_scale)
    return pl.pallas_call(
        body,
        grid=(n_slab,),
        in_specs=[
            pl.BlockSpec((_CODE_DIM, _TT), lambda i: (i, 0)),
            pl.BlockSpec((_NUM_CODES, _CODE_DIM), lambda i: (0, 0)),
        ],
        out_specs=[
            pl.BlockSpec((1, 1, _TT), lambda i: (i, 0, 0)),
            pl.BlockSpec((1, 1), lambda i: (0, 0)),
        ],
        out_shape=[
            jax.ShapeDtypeStruct((n_slab, 1, _TT), jnp.int32),
            jax.ShapeDtypeStruct((1, 1), jnp.float32),
        ],
    )(xt2d, emb)


def _make_sc_gather_t(n_tok, width):
    info = plsc.get_sparse_core_info()
    nc, ns, nl = info.num_cores, info.num_subcores, info.num_lanes
    nw = nc * ns
    b_per_w = n_tok // nw
    d_half = width // 2
    mesh = plsc.VectorSubcoreMesh(core_axis_name="c", subcore_axis_name="s")

    n_pass = 4
    d_pp = width // n_pass

    @functools.partial(
        pl.kernel,
        mesh=mesh,
        compiler_params=pltpu.CompilerParams(
            use_tc_tiling_on_sc=False, needs_layout_passes=False),
        out_type=jax.ShapeDtypeStruct((nw * width, b_per_w), jnp.float32),
        scratch_types=[
            pltpu.VMEM((width, _NUM_CODES), jnp.float32),
            pltpu.VMEM((b_per_w,), jnp.int32),
            pltpu.VMEM((d_pp, b_per_w), jnp.float32),
            pltpu.VMEM((d_pp, b_per_w), jnp.float32),
            pltpu.SemaphoreType.DMA,
            pltpu.SemaphoreType.DMA,
            pltpu.SemaphoreType.DMA,
        ],
    )
    def gather_k(tab_hbm, idx_hbm, out_hbm, tab_v, idx_v, o0, o1, si, s0, s1):
        wid = lax.axis_index("s") * nc + lax.axis_index("c")
        ht = pltpu.async_copy(tab_hbm, tab_v, si)
        hi = pltpu.async_copy(
            idx_hbm.at[pl.ds(wid * b_per_w, b_per_w)], idx_v, si)
        ht.wait()
        hi.wait()
        outs, sems = (o0, o1), (s0, s1)
        waits = [None] * n_pass
        for p in range(n_pass):
            if p >= 2:
                waits[p - 2].wait()
            out_v = outs[p % 2]
            rows = [jnp.full((nl,), p * d_pp + d, jnp.int32)
                    for d in range(d_pp)]

            def tok_body(t, _, out_v=out_v, rows=rows):
                iv = idx_v[pl.ds(t * nl, nl)]
                for d in range(d_pp):
                    out_v[d, pl.ds(t * nl, nl)] = plsc.load_gather(
                        tab_v, [rows[d], iv])
                return 0

            lax.fori_loop(0, b_per_w // nl, tok_body, 0, unroll=False)
            waits[p] = pltpu.async_copy(
                out_v, out_hbm.at[pl.ds(wid * width + p * d_pp, d_pp)],
                sems[p % 2])
        waits[n_pass - 2].wait()
        waits[n_pass - 1].wait()

    return gather_k


def kernel(z, embedding):
    n_tok = z.shape[0] * z.shape[1]
    scale = _COMMITMENT / (n_tok * _CODE_DIM)
    xt2d = jnp.transpose(z, (0, 2, 1)).reshape(-1, z.shape[1])
    idx3d, loss2d = _dist_argmin(xt2d, embedding, scale)
    indices = idx3d.reshape(n_tok)
    embt = jnp.transpose(embedding)
    zq_t = _make_sc_gather_t(n_tok, _CODE_DIM)(embt, indices)
    z_q = zq_t.reshape(z.shape[0], _CODE_DIM, z.shape[1]).transpose(0, 2, 1)
    return z_q, loss2d.reshape(()), indices
